# baseline (speedup 1.0000x reference)
import jax
import jax.numpy as jnp
from jax.experimental import pallas as pl

_STEP_CURRENT = 10
_N_HIST = _STEP_CURRENT + 1
_N_TARGET = 8
_PI = 3.141592653589793


def _wrap_rad(x):
    m = x + _PI
    m = m - (2.0 * _PI) * jnp.floor(m / (2.0 * _PI))
    return m - _PI


def _scene_kernel(feat_ref, static_ref, idx_ref, scal_ref, ofeat_ref, ostatic_ref):
    A = feat_ref.shape[1]
    P = _N_TARGET

    role = static_ref[0, :, 0, 3:6]
    w = jnp.sum(role, axis=1, keepdims=True)
    valid10 = feat_ref[0, :, 8, _STEP_CURRENT:_STEP_CURRENT + 1]
    w = w + valid10

    a_col = jax.lax.broadcasted_iota(jnp.int32, (A, 1), 0)
    key_col = w.astype(jnp.int32) * A + (A - 1 - a_col)
    key_row = jnp.transpose(key_col)
    gt = (key_row > key_col).astype(jnp.int32)
    rank_col = jnp.sum(gt, axis=1, keepdims=True)
    rank_row = jnp.transpose(rank_col)

    p_col = jax.lax.broadcasted_iota(jnp.int32, (P, 1), 0)
    sel = rank_row == p_col
    a_row = jax.lax.broadcasted_iota(jnp.int32, (P, A), 1)
    idx_col = jnp.sum(jnp.where(sel, a_row, 0), axis=1, keepdims=True)
    idx_ref[0] = idx_col

    for p in range(P):
        ip = jnp.sum(jnp.where(sel[p:p + 1, :], a_row[p:p + 1, :], 0))
        ofeat_ref[0, p] = feat_ref[0, ip]
        ostatic_ref[0, p] = static_ref[0, ip]

    G = ofeat_ref[0]
    col = G[:, :, _STEP_CURRENT:_STEP_CURRENT + 1]
    px = col[:, 0:1, :]
    py = col[:, 1:2, :]
    yaw0 = col[:, 6:7, :]
    c = jnp.cos(yaw0)
    s = jnp.sin(yaw0)

    dx = G[:, 0:1, :] - px
    dy = G[:, 1:2, :] - py
    vx = G[:, 2:3, :]
    vy = G[:, 3:4, :]
    ofeat_ref[0, :, 0:1, :] = dx * c + dy * s
    ofeat_ref[0, :, 1:2, :] = dy * c - dx * s
    ofeat_ref[0, :, 2:3, :] = vx * c + vy * s
    ofeat_ref[0, :, 3:4, :] = vy * c - vx * s
    ofeat_ref[0, :, 6:7, :] = _wrap_rad(G[:, 6:7, :] - yaw0)

    scal_ref[0, :, 0:1] = px[:, 0, :]
    scal_ref[0, :, 1:2] = py[:, 0, :]
    scal_ref[0, :, 2:3] = c[:, 0, :]
    scal_ref[0, :, 3:4] = s[:, 0, :]


def kernel(agent_valid, agent_pos, agent_vel, agent_spd, agent_acc,
           agent_yaw_bbox, agent_yaw_rate, agent_type, agent_role,
           agent_size, agent_cmd):
    S, T, A = agent_valid.shape
    P = _N_TARGET

    feat = jnp.concatenate([
        agent_pos, agent_vel, agent_spd, agent_acc,
        agent_yaw_bbox, agent_yaw_rate,
        agent_valid[..., None].astype(jnp.float32),
    ], axis=-1)
    feat = jnp.transpose(feat, (0, 2, 3, 1))

    static = jnp.concatenate([
        agent_type.astype(jnp.float32),
        agent_role.astype(jnp.float32),
        agent_size, agent_cmd,
    ], axis=-1)[:, :, None, :]

    out_shapes = (
        jax.ShapeDtypeStruct((S, P, 1), jnp.int32),
        jax.ShapeDtypeStruct((S, P, 4), jnp.float32),
        jax.ShapeDtypeStruct((S, P, 9, T), jnp.float32),
        jax.ShapeDtypeStruct((S, P, 1, 17), jnp.float32)
    )
    grid = (S,)
    out_idx, out_scal, out_feat, out_static = pl.pallas_call(
        _scene_kernel,
        grid=grid,
        in_specs=[
            pl.BlockSpec((1, A, 9, T), lambda s: (s, 0, 0, 0)),
            pl.BlockSpec((1, A, 1, 17), lambda s: (s, 0, 0, 0)),
        ],
        out_specs=(
            pl.BlockSpec((1, P, 1), lambda s: (s, 0, 0)),
            pl.BlockSpec((1, P, 4), lambda s: (s, 0, 0)),
            pl.BlockSpec((1, P, 9, T), lambda s: (s, 0, 0, 0)),
            pl.BlockSpec((1, P, 1, 17), lambda s: (s, 0, 0, 0)),
        ),
        out_shape=out_shapes,
    )(feat, static)

    target_indices = out_idx[:, :, 0]
    ref_pos = out_scal[:, :, None, 0:2]
    c = out_scal[:, :, 2]
    s = out_scal[:, :, 3]
    ref_rot = jnp.stack(
        [jnp.stack([c, -s], axis=-1), jnp.stack([s, c], axis=-1)], axis=-2)

    tr = jnp.transpose(out_feat, (0, 1, 3, 2))
    hist = tr[:, :, :_N_HIST]
    fut = tr[:, :, _N_HIST:]

    tgt_valid = hist[..., 8] > 0.5
    tgt_pos = hist[..., 0:2]
    tgt_vel = hist[..., 2:4]
    tgt_spd = hist[..., 4:5]
    tgt_acc = hist[..., 5:6]
    tgt_yaw = hist[..., 6:7]
    tgt_yaw_rate = hist[..., 7:8]

    gt_valid = fut[..., 8] > 0.5
    gt_pos = fut[..., 0:2]
    gt_vel = fut[..., 2:4]
    gt_spd = fut[..., 4:5]
    gt_yaw = fut[..., 6:7]

    st = out_static[:, :, 0, :]
    ref_type = st[..., 0:3] > 0.5
    ref_role = st[..., 3:6] > 0.5
    tgt_size = st[..., 6:9]
    gt_cmd = st[..., 9:17]

    return (target_indices, ref_pos, ref_rot, ref_type, ref_role,
            tgt_valid, tgt_pos, tgt_vel, tgt_spd, tgt_acc, tgt_yaw,
            tgt_yaw_rate, ref_type, ref_role, tgt_size,
            gt_valid, gt_pos, gt_spd, gt_vel, gt_yaw, gt_cmd)
